# baseline (speedup 1.0000x reference)
import functools

import jax
import jax.numpy as jnp
from jax import lax
from jax.experimental import pallas as pl
from jax.experimental.pallas import tpu as pltpu

B = 16384
C = 1000
MARGIN = 1.0

BR = 4096
GRID = B // BR


def _tc_hinge_body(x_ref, y_ref, out_ref):
    pi = pl.program_id(0)
    x = x_ref[...]
    yv = y_ref[0, 0, :]
    ycol = yv.reshape(BR, 1)
    col = lax.broadcasted_iota(jnp.int32, (BR, C), 1)
    oy = jnp.sum(jnp.where(col == ycol, x, 0.0), axis=1, keepdims=True)
    s = jnp.sum(jnp.maximum(x - oy + MARGIN, 0.0))

    @pl.when(pi == 0)
    def _init():
        out_ref[0, 0] = 0.0

    out_ref[0, 0] += s

    @pl.when(pi == GRID - 1)
    def _final():
        out_ref[0, 0] = (out_ref[0, 0] - float(B)) / float(B)


_tc_hinge = pl.pallas_call(
    _tc_hinge_body,
    grid=(GRID,),
    in_specs=[
        pl.BlockSpec((BR, C), lambda i: (i, 0)),
        pl.BlockSpec((1, 1, BR), lambda i: (i, 0, 0)),
    ],
    out_specs=pl.BlockSpec((1, 1), lambda i: (0, 0), memory_space=pltpu.SMEM),
    out_shape=jax.ShapeDtypeStruct((1, 1), jnp.float32),
)


def kernel(output, y):
    y3 = y.astype(jnp.int32).reshape(GRID, 1, BR)
    total = _tc_hinge(output, y3)
    return total[0, 0]
